# Initial kernel scaffold; baseline (speedup 1.0000x reference)
import jax
import jax.numpy as jnp
from jax.experimental import pallas as pl


def kernel(feats, params, neigh):
    raise NotImplementedError("write your pallas kernel here")


# speedup vs baseline: 1.7412x; 1.7412x over previous
import functools

import jax
import jax.numpy as jnp
from jax import lax
from jax.experimental import pallas as pl
from jax.experimental.pallas import tpu as pltpu
from jax.experimental.pallas import tpu_sc as plsc

_NC = 2
_NS = 16
_NW = _NC * _NS


def _sc_gather_call(table, idx2d, sub):
    nrows = idx2d.shape[0]
    c = table.shape[1]
    nblk = nrows // sub
    iters = -(-nblk // _NW)
    mesh = plsc.VectorSubcoreMesh(core_axis_name="c", subcore_axis_name="s")

    @functools.partial(
        pl.kernel,
        mesh=mesh,
        compiler_params=pltpu.CompilerParams(use_tc_tiling_on_sc=False),
        out_type=jax.ShapeDtypeStruct((nrows * 128, c), jnp.float32),
        scratch_types=[
            pltpu.VMEM((sub, 128), jnp.int32),
            pltpu.VMEM((sub * 128, c), jnp.float32),
            pltpu.SemaphoreType.DMA,
        ],
    )
    def k(table_hbm, idx_hbm, out_hbm, idx_v, rows_v, sem):
        wid = lax.axis_index("s") * _NC + lax.axis_index("c")

        def body(j, carry):
            b = wid + j * _NW

            @pl.when(b < nblk)
            def _():
                r0 = b * sub
                pltpu.sync_copy(idx_hbm.at[pl.ds(r0, sub)], idx_v)
                cps = [
                    pltpu.async_copy(
                        table_hbm.at[idx_v.at[s]],
                        rows_v.at[pl.ds(s * 128, 128)],
                        sem,
                    )
                    for s in range(sub)
                ]
                for cp in cps:
                    cp.wait()
                pltpu.sync_copy(rows_v, out_hbm.at[pl.ds(r0 * 128, sub * 128)])

            return carry

        lax.fori_loop(0, iters, body, 0)

    return k(table, idx2d)


def _gather_rows(x, nbmat):
    n_out, kk = nbmat.shape
    c = x.shape[1]
    b = n_out * kk
    sub = max(1, min(16, 512 // c))
    unit = 128 * sub
    bp = -(-b // unit) * unit
    idx = jnp.pad(nbmat.reshape(-1), (0, bp - b)).reshape(-1, 128)
    g = _sc_gather_call(x, idx, sub)
    return g[:b]


def _mm_body(g_ref, w_ref, z_ref):
    z_ref[...] = jnp.dot(g_ref[...], w_ref[...],
                         preferred_element_type=jnp.float32)


def _pick_tile(n, kc):
    for t in (512, 400, 320, 256, 200, 160, 128, 80, 64, 40, 32, 16, 8):
        if n % t == 0 and t * kc * 4 <= 4 * 1024 * 1024:
            return t
    return 8


def _matmul(g, w):
    n, kc = g.shape
    d = w.shape[1]
    t = _pick_tile(n, kc)
    return pl.pallas_call(
        _mm_body,
        grid=(n // t,),
        in_specs=[
            pl.BlockSpec((t, kc), lambda i: (i, 0)),
            pl.BlockSpec((kc, d), lambda i: (0, 0)),
        ],
        out_specs=pl.BlockSpec((t, d), lambda i: (i, 0)),
        out_shape=jax.ShapeDtypeStruct((n, d), jnp.float32),
    )(g, w)


def _conv(x, nbmat, wflat):
    n_out, kk = nbmat.shape
    g = _gather_rows(x, nbmat)
    return _matmul(g.reshape(n_out, kk * x.shape[1]), wflat)


def _norm(z, gm, bt, res=None, relu=True):
    n, d = z.shape
    gm2 = gm.reshape(1, d)
    bt2 = bt.reshape(1, d)

    def body(*refs):
        if res is None:
            z_ref, g_ref, b_ref, o_ref = refs
            r = None
        else:
            z_ref, g_ref, b_ref, r_ref, o_ref = refs
            r = r_ref[...]
        zz = z_ref[...]
        m = jnp.mean(zz, axis=0, keepdims=True)
        v = jnp.mean((zz - m) ** 2, axis=0, keepdims=True)
        y = (zz - m) * lax.rsqrt(v + 1e-5) * g_ref[...] + b_ref[...]
        if r is not None:
            y = y + r
        if relu:
            y = jnp.maximum(y, 0.0)
        o_ref[...] = y

    args = (z, gm2, bt2) if res is None else (z, gm2, bt2, res)
    return pl.pallas_call(
        body,
        out_shape=jax.ShapeDtypeStruct((n, d), jnp.float32),
    )(*args)


def _resblock(x, bp, nbmat):
    kk = nbmat.shape[1]
    c = x.shape[1]
    h = _norm(_conv(x, nbmat, bp['w1'].reshape(kk * c, c)),
              bp['g1'], bp['b1'], relu=True)
    return _norm(_conv(h, nbmat, bp['w2'].reshape(kk * c, c)),
                 bp['g2'], bp['b2'], res=x, relu=True)


def _final_body(x_ref, w1_ref, w2_ref, b_ref, o_ref):
    h = jnp.maximum(
        jnp.dot(x_ref[...], w1_ref[...], preferred_element_type=jnp.float32),
        0.0)
    o_ref[...] = jnp.dot(h, w2_ref[...],
                         preferred_element_type=jnp.float32) + b_ref[...]


def _final(x, w1, w2, b2):
    n = x.shape[0]
    d = w2.shape[1]
    return pl.pallas_call(
        _final_body,
        out_shape=jax.ShapeDtypeStruct((n, d), jnp.float32),
    )(x, w1, w2, b2.reshape(1, d))


def kernel(feats, params, neigh):
    p, nb = params, neigh

    x0 = jnp.pad(feats, ((0, 0), (0, 13)))
    w1 = jnp.pad(p['conv1'], ((0, 0), (0, 13), (0, 0))).reshape(125 * 16, 32)

    a = _norm(_conv(x0, nb['n1k5'], w1), p['n1g'], p['n1b'], relu=False)
    s1 = _resblock(a, p['block1'], nb['n1'])

    z = _conv(s1, nb['d1'], p['conv2'].reshape(27 * 32, 64))
    s2 = _resblock(_norm(z, p['n2g'], p['n2b'], relu=False),
                   p['block2'], nb['n2'])

    z = _conv(s2, nb['d2'], p['conv3'].reshape(27 * 64, 128))
    s4 = _resblock(_norm(z, p['n3g'], p['n3b'], relu=False),
                   p['block3'], nb['n3'])

    z = _conv(s4, nb['d3'], p['conv4'].reshape(27 * 128, 256))
    s8 = _resblock(_norm(z, p['n4g'], p['n4b'], relu=False),
                   p['block4'], nb['n4'])

    z = _conv(s8, nb['u3'], p['conv4tr'].reshape(27 * 256, 128))
    t = _resblock(_norm(z, p['n4tg'], p['n4tb'], relu=False),
                  p['block4tr'], nb['n3'])

    cat = jnp.concatenate([t, s4], axis=1)
    z = _conv(cat, nb['u2'], p['conv3tr'].reshape(27 * 256, 64))
    t = _resblock(_norm(z, p['n3tg'], p['n3tb'], relu=False),
                  p['block3tr'], nb['n2'])

    cat = jnp.concatenate([t, s2], axis=1)
    z = _conv(cat, nb['u1'], p['conv2tr'].reshape(27 * 128, 64))
    t = _resblock(_norm(z, p['n2tg'], p['n2tb'], relu=False),
                  p['block2tr'], nb['n1'])

    cat = jnp.concatenate([t, s1], axis=1)
    return _final(cat, p['conv1tr'], p['finalW'], p['finalb'])
